# Initial kernel scaffold; baseline (speedup 1.0000x reference)
import jax
import jax.numpy as jnp
from jax.experimental import pallas as pl


def kernel(x, W1, b1, W2, b2, k):
    raise NotImplementedError("write your pallas kernel here")


# speedup vs baseline: 1.3945x; 1.3945x over previous
import functools

import jax
import jax.numpy as jnp
from jax.experimental import pallas as pl
from jax.experimental.pallas import tpu as pltpu

_BLK = 1024
_EPS = 1e-6
_ENT_W = 0.01


def _tg_kernel(x_ref, w1_ref, b1_ref, w2_ref, b2_ref, t_ref,
               y_ref, s_ref, p_ref, aux_ref,
               p_acc, ent_acc, bg_acc, lr_acc):
    b = pl.program_id(0)
    i = pl.program_id(1)
    nb = pl.num_programs(1) - 1
    n_tok = nb * _BLK

    @pl.when(i == 0)
    def _zero_batch():
        p_acc[...] = jnp.zeros_like(p_acc)
        bg_acc[...] = jnp.zeros_like(bg_acc)

    @pl.when((b == 0) & (i == 0))
    def _zero_global():
        ent_acc[...] = jnp.zeros_like(ent_acc)
        lr_acc[...] = jnp.zeros_like(lr_acc)

    @pl.when(i < nb)
    def _body():
        x2d = x_ref[0]
        h = jnp.dot(x2d, w1_ref[...], preferred_element_type=jnp.float32)
        h = h + b1_ref[...]
        h = 0.5 * h * (1.0 + jax.lax.erf(h * 0.7071067811865476))
        s_col = jnp.dot(h, w2_ref[...], preferred_element_type=jnp.float32)
        s_col = s_col + b2_ref[...]
        p_col = jax.nn.sigmoid(s_col)
        y_blk = x2d * p_col
        y_ref[...] = y_blk[None]
        s_ref[...] = s_col[None]
        p_ref[...] = p_col[None]
        bg_acc[...] += jnp.sum(x2d - y_blk, axis=0, keepdims=True)
        p_acc[...] += p_col
        ent_acc[...] += -(p_col * jnp.log(p_col + _EPS)
                          + (1.0 - p_col) * jnp.log(1.0 - p_col + _EPS))

    @pl.when(i == nb)
    def _finalize():
        psum = jnp.sum(p_acc[...], axis=(0, 1), keepdims=True)
        bgw = jnp.maximum(n_tok - psum, _EPS)
        y_ref[0, 0:1, :] = bg_acc[...] / bgw
        ratio = psum / n_tok
        lr = lr_acc[...] + (ratio - t_ref[...]) ** 2
        lr_acc[...] = lr

        @pl.when(b == pl.num_programs(0) - 1)
        def _aux():
            ent = jnp.sum(ent_acc[...], axis=(0, 1), keepdims=True)
            nb_total = pl.num_programs(0) * n_tok
            aux_ref[...] = (lr / pl.num_programs(0)
                            + _ENT_W * ent / nb_total)


@functools.partial(jax.jit, static_argnames=())
def kernel(x, W1, b1, W2, b2, k):
    B, N, D = x.shape
    H = W1.shape[1]
    nb = N // _BLK
    kc = jnp.clip(jnp.asarray(k), 0, N)
    t = (kc.astype(jnp.float32) / float(N)).reshape(1, 1)

    grid = (B, nb + 1)
    y, s3, p3, aux = pl.pallas_call(
        _tg_kernel,
        grid=grid,
        in_specs=[
            pl.BlockSpec((1, _BLK, D),
                         lambda b, i: (b, jnp.minimum(i, nb - 1), 0)),
            pl.BlockSpec((D, H), lambda b, i: (0, 0)),
            pl.BlockSpec((1, H), lambda b, i: (0, 0)),
            pl.BlockSpec((H, 1), lambda b, i: (0, 0)),
            pl.BlockSpec((1, 1), lambda b, i: (0, 0)),
            pl.BlockSpec((1, 1), lambda b, i: (0, 0)),
        ],
        out_specs=[
            pl.BlockSpec((1, _BLK, D), lambda b, i: (b, i, 0)),
            pl.BlockSpec((1, _BLK, 1),
                         lambda b, i: (b, jnp.minimum(i, nb - 1), 0)),
            pl.BlockSpec((1, _BLK, 1),
                         lambda b, i: (b, jnp.minimum(i, nb - 1), 0)),
            pl.BlockSpec((1, 1), lambda b, i: (0, 0)),
        ],
        out_shape=[
            jax.ShapeDtypeStruct((B, N + 1, D), jnp.float32),
            jax.ShapeDtypeStruct((B, N, 1), jnp.float32),
            jax.ShapeDtypeStruct((B, N, 1), jnp.float32),
            jax.ShapeDtypeStruct((1, 1), jnp.float32),
        ],
        scratch_shapes=[
            pltpu.VMEM((_BLK, 1), jnp.float32),
            pltpu.VMEM((_BLK, 1), jnp.float32),
            pltpu.VMEM((1, D), jnp.float32),
            pltpu.VMEM((1, 1), jnp.float32),
        ],
        compiler_params=pltpu.CompilerParams(
            dimension_semantics=("arbitrary", "arbitrary"),
        ),
    )(x, W1, b1.reshape(1, H), W2, b2.reshape(1, 1), t)

    return (y, aux[0, 0], s3[..., 0], p3[..., 0])
